# Initial kernel scaffold; baseline (speedup 1.0000x reference)
import functools
import jax
import jax.numpy as jnp
from jax import lax
from jax.experimental import pallas as pl
from jax.experimental.pallas import tpu as pltpu
from jax.experimental.pallas import tpu_sc as plsc

N = 10000
E = 320000
D = 128
XI = 64
H = 4
O = 128
DZ = D + XI
NPAD = 10240
SCALE = float(DZ) ** -0.5

NC = 2
NS = 16
NW = NC * NS
EPW = E // NW
CH = 80
NCHUNK = EPW // CH
ROWS_PER_SUB = NPAD // NS

_f32 = jnp.float32
_i32 = jnp.int32


def _qkv_body(x_ref, emb_ref, w_ref, b_ref, out_ref):
    w = w_ref[...]
    wsum = jnp.sum(w[:XI, :], axis=0, keepdims=True)
    out_ref[...] = (x_ref[...] * wsum
                    + jnp.dot(emb_ref[...], w[XI:, :],
                              preferred_element_type=_f32)
                    + b_ref[...])


def _qkv_call(xp, embp, qkv_W, qkv_b):
    BQ = 512
    grid = (NPAD // BQ,)
    return pl.pallas_call(
        _qkv_body,
        grid=grid,
        in_specs=[
            pl.BlockSpec((BQ, 1), lambda i: (i, 0)),
            pl.BlockSpec((BQ, D), lambda i: (i, 0)),
            pl.BlockSpec((DZ, 3 * DZ), lambda i: (0, 0)),
            pl.BlockSpec((1, 3 * DZ), lambda i: (0, 0)),
        ],
        out_specs=pl.BlockSpec((BQ, 3 * DZ), lambda i: (i, 0)),
        out_shape=jax.ShapeDtypeStruct((NPAD, 3 * DZ), _f32),
    )(xp, embp, qkv_W, qkv_b.reshape(1, -1))


BQF = 256
BKF = 1024
NKF = NPAD // BKF


def _head_fold(Wm, attnm):
    cols = [jnp.sum(Wm[:, h * O:(h + 1) * O] * attnm[h:h + 1, :],
                    axis=1, keepdims=True) for h in range(H)]
    return jnp.concatenate(cols, axis=1)


def _bias_fold(bm, attnm):
    cols = [jnp.sum(bm[:, h * O:(h + 1) * O] * attnm[h:h + 1, :],
                    axis=1, keepdims=True) for h in range(H)]
    return jnp.concatenate(cols, axis=1)


def _flash_body(q_ref, k_ref, v_ref, ei_ref,
                projW_ref, projb_ref, linW_ref, linb_ref, lng_ref, lnb_ref,
                Wt_ref, bt_ref, Wi_ref, bi_ref, Wj_ref, bj_ref, attn_ref,
                emb_out, xtn_out, aij_out,
                m_scr, l_scr, acc_scr):
    j = pl.program_id(1)

    @pl.when(j == 0)
    def _():
        m_scr[...] = jnp.full((BQF, 128), -3e38, _f32)
        l_scr[...] = jnp.zeros((BQF, 128), _f32)
        acc_scr[...] = jnp.zeros((BQF, DZ), _f32)

    q = q_ref[...]
    k = k_ref[...]
    logits = lax.dot_general(q, k, (((1,), (1,)), ((), ())),
                             preferred_element_type=_f32) * SCALE
    col = j * BKF + lax.broadcasted_iota(_i32, (BQF, BKF), 1)
    logits = jnp.where(col < N, logits, -3e38)

    m_prev = m_scr[:, 0:1]
    l_prev = l_scr[:, 0:1]
    m_new = jnp.maximum(m_prev, jnp.max(logits, axis=1, keepdims=True))
    alpha = jnp.exp(m_prev - m_new)
    p = jnp.exp(logits - m_new)
    l_new = l_prev * alpha + jnp.sum(p, axis=1, keepdims=True)
    acc_new = acc_scr[...] * alpha + jnp.dot(p, v_ref[...],
                                             preferred_element_type=_f32)
    m_scr[...] = jnp.broadcast_to(m_new, (BQF, 128))
    l_scr[...] = jnp.broadcast_to(l_new, (BQF, 128))
    acc_scr[...] = acc_new

    @pl.when(j == NKF - 1)
    def _():
        o = acc_new / l_new
        o2 = jnp.dot(o, projW_ref[...], preferred_element_type=_f32) \
            + projb_ref[...]
        hh = jnp.dot(o2, linW_ref[...], preferred_element_type=_f32) \
            + linb_ref[...]
        mu = jnp.mean(hh, axis=1, keepdims=True)
        var = jnp.mean((hh - mu) * (hh - mu), axis=1, keepdims=True)
        hln = (hh - mu) * lax.rsqrt(var + 1e-5) * lng_ref[...] + lnb_ref[...]
        hs = hln * jax.nn.sigmoid(hln)
        embv = hs + ei_ref[...]
        emb_out[...] = embv
        xtn_out[...] = jnp.dot(embv, Wt_ref[...],
                               preferred_element_type=_f32) + bt_ref[...]
        attnm = attn_ref[...]
        wA = jnp.concatenate([_head_fold(Wi_ref[...], attnm),
                              _head_fold(Wj_ref[...], attnm)], axis=1)
        cA = jnp.concatenate([_bias_fold(bi_ref[...], attnm),
                              _bias_fold(bj_ref[...], attnm)], axis=1)
        aij8 = jnp.dot(embv, wA, preferred_element_type=_f32) + cA
        aij_out[...] = jnp.concatenate(
            [aij8, jnp.zeros((BQF, 8), _f32)], axis=1)


def _flash_call(qkv, embp, proj_W, proj_b, lin_W, lin_b, ln_g, ln_b,
                Wt, bt, Wi, bi, Wj, bj, attn):
    grid = (NPAD // BQF, NKF)
    full = lambda r, c: pl.BlockSpec((r, c), lambda i, j: (0, 0))
    return pl.pallas_call(
        _flash_body,
        grid=grid,
        in_specs=[
            pl.BlockSpec((BQF, DZ), lambda i, j: (i, 0)),
            pl.BlockSpec((BKF, DZ), lambda i, j: (j, 1)),
            pl.BlockSpec((BKF, DZ), lambda i, j: (j, 2)),
            pl.BlockSpec((BQF, D), lambda i, j: (i, 0)),
            full(DZ, DZ), full(1, DZ), full(DZ, D), full(1, D),
            full(1, D), full(1, D),
            full(D, H * O), full(1, H * O),
            full(D, H * O), full(1, H * O),
            full(D, H * O), full(1, H * O),
            full(H, O),
        ],
        out_specs=[
            pl.BlockSpec((BQF, D), lambda i, j: (i, 0)),
            pl.BlockSpec((BQF, H * O), lambda i, j: (i, 0)),
            pl.BlockSpec((BQF, 16), lambda i, j: (i, 0)),
        ],
        out_shape=[
            jax.ShapeDtypeStruct((NPAD, D), _f32),
            jax.ShapeDtypeStruct((NPAD, H * O), _f32),
            jax.ShapeDtypeStruct((NPAD, 16), _f32),
        ],
        scratch_shapes=[
            pltpu.VMEM((BQF, 128), _f32),
            pltpu.VMEM((BQF, 128), _f32),
            pltpu.VMEM((BQF, DZ), _f32),
        ],
        compiler_params=pltpu.CompilerParams(
            dimension_semantics=("parallel", "arbitrary")),
    )(qkv, qkv, qkv, embp, proj_W, proj_b.reshape(1, -1), lin_W,
      lin_b.reshape(1, -1), ln_g.reshape(1, -1), ln_b.reshape(1, -1),
      Wt, bt.reshape(1, -1), Wi, bi.reshape(1, -1), Wj, bj.reshape(1, -1),
      attn)


def _tables_body(pa_ref, pb_ref, skW_ref, skb_ref,
                 Wt_ref, bt_ref, Wi_ref, bi_ref, Wj_ref, bj_ref, attn_ref,
                 s0_out, xtn_out, aij_out):
    e1 = pa_ref[...] + pb_ref[...]
    s0_out[...] = jnp.dot(e1, skW_ref[...],
                          preferred_element_type=_f32) + skb_ref[...]
    xtn_out[...] = jnp.dot(e1, Wt_ref[...],
                           preferred_element_type=_f32) + bt_ref[...]
    attnm = attn_ref[...]
    wA = jnp.concatenate([_head_fold(Wi_ref[...], attnm),
                          _head_fold(Wj_ref[...], attnm)], axis=1)
    cA = jnp.concatenate([_bias_fold(bi_ref[...], attnm),
                          _bias_fold(bj_ref[...], attnm)], axis=1)
    aij8 = jnp.dot(e1, wA, preferred_element_type=_f32) + cA
    aij_out[...] = jnp.concatenate([aij8, jnp.zeros((256, 8), _f32)], axis=1)


def _tables_call(pa, pb, skW, skb, Wt, bt, Wi, bi, Wj, bj, attn):
    BQ = 256
    grid = (NPAD // BQ,)
    full = lambda r, c: pl.BlockSpec((r, c), lambda i: (0, 0))
    blk = lambda c: pl.BlockSpec((BQ, c), lambda i: (i, 0))
    return pl.pallas_call(
        _tables_body,
        grid=grid,
        in_specs=[
            blk(D), blk(D),
            full(O, O), full(1, O),
            full(D, H * O), full(1, H * O),
            full(D, H * O), full(1, H * O),
            full(D, H * O), full(1, H * O),
            full(H, O),
        ],
        out_specs=[blk(D), blk(H * O), blk(16)],
        out_shape=[
            jax.ShapeDtypeStruct((NPAD, D), _f32),
            jax.ShapeDtypeStruct((NPAD, H * O), _f32),
            jax.ShapeDtypeStruct((NPAD, 16), _f32),
        ],
    )(pa, pb, skW, skb.reshape(1, -1), Wt, bt.reshape(1, -1),
      Wi, bi.reshape(1, -1), Wj, bj.reshape(1, -1), attn)


def _final_body(pa_ref, pb_ref, s0_ref, skW_ref, skb_ref, g_ref, b_ref,
                out_ref):
    e2 = pa_ref[...] + pb_ref[...]
    s1 = jnp.dot(e2, skW_ref[...], preferred_element_type=_f32) + skb_ref[...]
    out = e2 + (s0_ref[...] + s1) * 0.5
    mu = jnp.mean(out, axis=1, keepdims=True)
    var = jnp.mean((out - mu) * (out - mu), axis=1, keepdims=True)
    out_ref[...] = (out - mu) * lax.rsqrt(var + 1e-5) * g_ref[...] + b_ref[...]


def _final_call(pa, pb, s0, skW, skb, nf_g, nf_b):
    BQ = 256
    grid = (NPAD // BQ,)
    full = lambda r, c: pl.BlockSpec((r, c), lambda i: (0, 0))
    blk = lambda c: pl.BlockSpec((BQ, c), lambda i: (i, 0))
    return pl.pallas_call(
        _final_body,
        grid=grid,
        in_specs=[blk(D), blk(D), blk(D), full(O, O), full(1, O),
                  full(1, O), full(1, O)],
        out_specs=blk(D),
        out_shape=jax.ShapeDtypeStruct((NPAD, D), _f32),
    )(pa, pb, s0, skW, skb.reshape(1, -1), nf_g.reshape(1, -1),
      nf_b.reshape(1, -1))


_mesh = plsc.VectorSubcoreMesh(core_axis_name="c", subcore_axis_name="s")


def _edge_stats_body(src_hbm, dst_hbm, ea_hbm, aij_hbm,
                     m_hbm, z_hbm,
                     srcv, dstv, eav, gdv, gsv, row16, sem):
    cid = lax.axis_index("c")
    sid = lax.axis_index("s")
    wid = cid * NS + sid
    iota = lax.iota(_i32, 16)
    sh4 = jnp.bitwise_and(iota + 4, 15)

    def chunk_body(c, carry):
        m_acc, z_acc = carry
        base = wid * EPW + c * CH
        pltpu.sync_copy(src_hbm.at[pl.ds(base, CH)], srcv)
        pltpu.sync_copy(dst_hbm.at[pl.ds(base, CH)], dstv)
        pltpu.sync_copy(ea_hbm.at[pl.ds(base, CH)], eav)
        pltpu.async_copy(aij_hbm.at[dstv], gdv, sem).wait()
        pltpu.async_copy(aij_hbm.at[srcv], gsv, sem).wait()

        def ebody(e, ec):
            m_a, z_a = ec
            fe = jnp.full((16,), e, _i32)
            gd = plsc.load_gather(gdv, [fe, iota])
            gs4 = plsc.load_gather(gsv, [fe, sh4])
            eas = plsc.load_gather(eav, [fe])
            s = (gd + gs4) * eas
            m_n = jnp.maximum(m_a, s)
            z_n = z_a * jnp.exp(m_a - m_n) + jnp.exp(s - m_n)
            return m_n, z_n

        return lax.fori_loop(0, CH, ebody, (m_acc, z_acc))

    m0 = jnp.full((16,), -3e38, _f32)
    z0 = jnp.zeros((16,), _f32)
    m_f, z_f = lax.fori_loop(0, NCHUNK, chunk_body, (m0, z0))
    row16[...] = m_f
    pltpu.sync_copy(row16, m_hbm.at[wid])
    row16[...] = z_f
    pltpu.sync_copy(row16, z_hbm.at[wid])


def _edge_stats_call(src, dst, ea, aij):
    kfn = pl.kernel(
        _edge_stats_body,
        out_type=[jax.ShapeDtypeStruct((NW, 16), _f32),
                  jax.ShapeDtypeStruct((NW, 16), _f32)],
        mesh=_mesh,
        scratch_types=[
            pltpu.VMEM((CH,), _i32),
            pltpu.VMEM((CH,), _i32),
            pltpu.VMEM((CH,), _f32),
            pltpu.VMEM((CH, 16), _f32),
            pltpu.VMEM((CH, 16), _f32),
            pltpu.VMEM((16,), _f32),
            pltpu.SemaphoreType.DMA,
        ],
    )
    return kfn(src, dst, ea, aij)


def _edge_aggr_body(src_hbm, dst_hbm, ea_hbm, aij_hbm, xtn_hbm,
                    m_hbm, z_hbm, zrows_hbm,
                    part_hbm,
                    srcv, dstv, eav, gdv, gsv, xrv, moutv, mzv, abv,
                    acc_sh, sem):
    cid = lax.axis_index("c")
    sid = lax.axis_index("s")
    wid = cid * NS + sid
    iota = lax.iota(_i32, 16)
    sh4 = jnp.bitwise_and(iota + 4, 15)

    pltpu.sync_copy(zrows_hbm, acc_sh.at[pl.ds(sid * ROWS_PER_SUB,
                                               ROWS_PER_SUB)])

    pltpu.sync_copy(m_hbm, mzv.at[pl.ds(0, NW)])
    pltpu.sync_copy(z_hbm, mzv.at[pl.ds(NW, NW)])

    def mfold(w, m_a):
        return jnp.maximum(m_a, plsc.load_gather(
            mzv, [jnp.full((16,), w, _i32), iota]))
    M = lax.fori_loop(0, NW, mfold, jnp.full((16,), -3e38, _f32))

    def zfold(w, z_a):
        fw = jnp.full((16,), w, _i32)
        mw = plsc.load_gather(mzv, [fw, iota])
        zw = plsc.load_gather(mzv, [fw + NW, iota])
        return z_a + zw * jnp.exp(mw - M)
    Z = lax.fori_loop(0, NW, zfold, jnp.zeros((16,), _f32))
    invZ = 1.0 / Z

    plsc.subcore_barrier()

    def chunk_body(c, _):
        base = wid * EPW + c * CH
        pltpu.sync_copy(src_hbm.at[pl.ds(base, CH)], srcv)
        pltpu.sync_copy(dst_hbm.at[pl.ds(base, CH)], dstv)
        pltpu.sync_copy(ea_hbm.at[pl.ds(base, CH)], eav)
        pltpu.async_copy(aij_hbm.at[dstv], gdv, sem).wait()
        pltpu.async_copy(aij_hbm.at[srcv], gsv, sem).wait()
        pltpu.async_copy(xtn_hbm.at[srcv], xrv, sem).wait()

        def ebody(e, _e):
            fe = jnp.full((16,), e, _i32)
            gd = plsc.load_gather(gdv, [fe, iota])
            gs4 = plsc.load_gather(gsv, [fe, sh4])
            eas = plsc.load_gather(eav, [fe])
            s = (gd + gs4) * eas
            abv[...] = jnp.exp(s - M) * invZ
            acc = [None] * 8
            for h in range(H):
                w = plsc.load_gather(abv, [jnp.full((16,), h, _i32)])
                for ct in range(8):
                    xv = plsc.load_gather(xrv, [fe, iota + (h * O + ct * 16)])
                    acc[ct] = w * xv if h == 0 else acc[ct] + w * xv
            for ct in range(8):
                plsc.store_scatter(moutv, [fe, iota + ct * 16], acc[ct])
            return 0

        lax.fori_loop(0, CH, ebody, 0)
        pltpu.sync_copy(moutv, acc_sh.at[dstv], add=True)
        return 0

    lax.fori_loop(0, NCHUNK, chunk_body, 0)

    plsc.subcore_barrier()
    pltpu.sync_copy(acc_sh.at[pl.ds(sid * ROWS_PER_SUB, ROWS_PER_SUB)],
                    part_hbm.at[cid, pl.ds(sid * ROWS_PER_SUB,
                                           ROWS_PER_SUB)])


def _edge_aggr_call(src, dst, ea, aij, xtn, m_p, z_p, zrows):
    kfn = pl.kernel(
        _edge_aggr_body,
        out_type=jax.ShapeDtypeStruct((NC, NPAD, D), _f32),
        mesh=_mesh,
        scratch_types=[
            pltpu.VMEM((CH,), _i32),
            pltpu.VMEM((CH,), _i32),
            pltpu.VMEM((CH,), _f32),
            pltpu.VMEM((CH, 16), _f32),
            pltpu.VMEM((CH, 16), _f32),
            pltpu.VMEM((CH, H * O), _f32),
            pltpu.VMEM((CH, D), _f32),
            pltpu.VMEM((2 * NW, 16), _f32),
            pltpu.VMEM((16,), _f32),
            pltpu.VMEM_SHARED((NPAD, D), _f32),
            pltpu.SemaphoreType.DMA,
        ],
    )
    return kfn(src, dst, ea, aij, xtn, m_p, z_p, zrows)


def _edge_layer(src, dst, ea, aij, xtn, zrows):
    m_p, z_p = _edge_stats_call(src, dst, ea, aij)
    part = _edge_aggr_call(src, dst, ea, aij, xtn, m_p, z_p, zrows)
    return part[0], part[1]


@jax.jit
def kernel(x, ids, edge_index, edge_attr, embed, qkv_W, qkv_b, proj_W,
           proj_b, lin_W, lin_b, ln_g, ln_b,
           Wi0, bi0, Wj0, bj0, Wt0, bt0, attn0, skW0, skb0,
           Wi1, bi1, Wj1, bj1, Wt1, bt1, attn1, skW1, skb1,
           nf_g, nf_b):
    Ei = jnp.take(embed, ids, axis=0)
    xp = jnp.pad(x, ((0, NPAD - N), (0, 0)))
    embp = jnp.pad(Ei, ((0, NPAD - N), (0, 0)))
    src = edge_index[0].astype(_i32)
    dst = edge_index[1].astype(_i32)
    zrows = jnp.zeros((ROWS_PER_SUB, D), _f32)

    qkv = _qkv_call(xp, embp, qkv_W, qkv_b)
    emb, xtn0, aij0 = _flash_call(qkv, embp, proj_W, proj_b, lin_W, lin_b,
                                  ln_g, ln_b, Wt0, bt0, Wi0, bi0, Wj0, bj0,
                                  attn0)
    pa0, pb0 = _edge_layer(src, dst, edge_attr, aij0, xtn0, zrows)
    s0, xtn1, aij1 = _tables_call(pa0, pb0, skW0, skb0, Wt1, bt1,
                                  Wi1, bi1, Wj1, bj1, attn1)
    pa1, pb1 = _edge_layer(src, dst, edge_attr, aij1, xtn1, zrows)
    outp = _final_call(pa1, pb1, s0, skW1, skb1, nf_g, nf_b)
    return outp[:N]


# speedup vs baseline: 1.3131x; 1.3131x over previous
import functools
import jax
import jax.numpy as jnp
from jax import lax
from jax.experimental import pallas as pl
from jax.experimental.pallas import tpu as pltpu
from jax.experimental.pallas import tpu_sc as plsc

N = 10000
E = 320000
D = 128
XI = 64
H = 4
O = 128
DZ = D + XI
NPAD = 10240
SCALE = float(DZ) ** -0.5

NC = 2
NS = 16
NW = NC * NS
EPW = E // NW
CH = 80
NCHUNK = EPW // CH
ROWS_PER_SUB = NPAD // NS
HALF = NPAD // 2
HALF_PAD = 5248
EPW2 = E // NS
NCHUNK2 = EPW2 // CH
RPS2 = HALF_PAD // NS

_f32 = jnp.float32
_i32 = jnp.int32


def _qkv_body(x_ref, emb_ref, wq_ref, wk_ref, wv_ref, b_ref,
              q_out, k_out, v_out):
    x = x_ref[...]
    emb = emb_ref[...]
    b = b_ref[...]
    for wr, out, col in ((wq_ref, q_out, 0), (wk_ref, k_out, 1),
                         (wv_ref, v_out, 2)):
        w = wr[...]
        wsum = jnp.sum(w[:XI, :], axis=0, keepdims=True)
        out[...] = (x * wsum
                    + jnp.dot(emb, w[XI:, :], preferred_element_type=_f32)
                    + b[col:col + 1, :])


def _qkv_call(xp, embp, qkv_W, qkv_b):
    BQ = 512
    grid = (NPAD // BQ,)
    out_blk = pl.BlockSpec((BQ, DZ), lambda i: (i, 0))
    return pl.pallas_call(
        _qkv_body,
        grid=grid,
        in_specs=[
            pl.BlockSpec((BQ, 1), lambda i: (i, 0)),
            pl.BlockSpec((BQ, D), lambda i: (i, 0)),
            pl.BlockSpec((DZ, DZ), lambda i: (0, 0)),
            pl.BlockSpec((DZ, DZ), lambda i: (0, 0)),
            pl.BlockSpec((DZ, DZ), lambda i: (0, 0)),
            pl.BlockSpec((3, DZ), lambda i: (0, 0)),
        ],
        out_specs=[out_blk, out_blk, out_blk],
        out_shape=[jax.ShapeDtypeStruct((NPAD, DZ), _f32)] * 3,
    )(xp, embp, qkv_W[:, :DZ], qkv_W[:, DZ:2 * DZ], qkv_W[:, 2 * DZ:],
      qkv_b.reshape(3, DZ))


BQF = 256
BKF = 1024
NKF = NPAD // BKF


def _head_fold(Wm, attnm):
    cols = [jnp.sum(Wm[:, h * O:(h + 1) * O] * attnm[h:h + 1, :],
                    axis=1, keepdims=True) for h in range(H)]
    return jnp.concatenate(cols, axis=1)


def _bias_fold(bm, attnm):
    cols = [jnp.sum(bm[:, h * O:(h + 1) * O] * attnm[h:h + 1, :],
                    axis=1, keepdims=True) for h in range(H)]
    return jnp.concatenate(cols, axis=1)


def _flash_body(q_ref, k_ref, v_ref, ei_ref,
                projW_ref, projb_ref, linW_ref, linb_ref, lng_ref, lnb_ref,
                Wt_ref, bt_ref, Wi_ref, bi_ref, Wj_ref, bj_ref, attn_ref,
                emb_out, xtn_out, aij_out,
                m_scr, l_scr, acc_scr):
    j = pl.program_id(1)

    @pl.when(j == 0)
    def _():
        m_scr[...] = jnp.full((BQF, 128), -3e38, _f32)
        l_scr[...] = jnp.zeros((BQF, 128), _f32)
        acc_scr[...] = jnp.zeros((BQF, DZ), _f32)

    q = q_ref[...]
    k = k_ref[...]
    logits = lax.dot_general(q, k, (((1,), (1,)), ((), ())),
                             preferred_element_type=_f32) * SCALE
    col = j * BKF + lax.broadcasted_iota(_i32, (BQF, BKF), 1)
    logits = jnp.where(col < N, logits, -3e38)

    m_prev = m_scr[:, 0:1]
    l_prev = l_scr[:, 0:1]
    m_new = jnp.maximum(m_prev, jnp.max(logits, axis=1, keepdims=True))
    alpha = jnp.exp(m_prev - m_new)
    p = jnp.exp(logits - m_new)
    l_new = l_prev * alpha + jnp.sum(p, axis=1, keepdims=True)
    acc_new = acc_scr[...] * alpha + jnp.dot(p, v_ref[...],
                                             preferred_element_type=_f32)
    m_scr[...] = jnp.broadcast_to(m_new, (BQF, 128))
    l_scr[...] = jnp.broadcast_to(l_new, (BQF, 128))
    acc_scr[...] = acc_new

    @pl.when(j == NKF - 1)
    def _():
        o = acc_new / l_new
        o2 = jnp.dot(o, projW_ref[...], preferred_element_type=_f32) \
            + projb_ref[...]
        hh = jnp.dot(o2, linW_ref[...], preferred_element_type=_f32) \
            + linb_ref[...]
        mu = jnp.mean(hh, axis=1, keepdims=True)
        var = jnp.mean((hh - mu) * (hh - mu), axis=1, keepdims=True)
        hln = (hh - mu) * lax.rsqrt(var + 1e-5) * lng_ref[...] + lnb_ref[...]
        hs = hln * jax.nn.sigmoid(hln)
        embv = hs + ei_ref[...]
        emb_out[...] = embv
        xtn_out[...] = jnp.dot(embv, Wt_ref[...],
                               preferred_element_type=_f32) + bt_ref[...]
        attnm = attn_ref[...]
        wA = jnp.concatenate([_head_fold(Wi_ref[...], attnm),
                              _head_fold(Wj_ref[...], attnm)], axis=1)
        cA = jnp.concatenate([_bias_fold(bi_ref[...], attnm),
                              _bias_fold(bj_ref[...], attnm)], axis=1)
        aij8 = jnp.dot(embv, wA, preferred_element_type=_f32) + cA
        aij_out[...] = jnp.concatenate(
            [aij8, jnp.zeros((BQF, 8), _f32)], axis=1)


def _flash_call(q, k, v, embp, proj_W, proj_b, lin_W, lin_b, ln_g, ln_b,
                Wt, bt, Wi, bi, Wj, bj, attn):
    grid = (NPAD // BQF, NKF)
    full = lambda r, c: pl.BlockSpec((r, c), lambda i, j: (0, 0))
    return pl.pallas_call(
        _flash_body,
        grid=grid,
        in_specs=[
            pl.BlockSpec((BQF, DZ), lambda i, j: (i, 0)),
            pl.BlockSpec((BKF, DZ), lambda i, j: (j, 0)),
            pl.BlockSpec((BKF, DZ), lambda i, j: (j, 0)),
            pl.BlockSpec((BQF, D), lambda i, j: (i, 0)),
            full(DZ, DZ), full(1, DZ), full(DZ, D), full(1, D),
            full(1, D), full(1, D),
            full(D, H * O), full(1, H * O),
            full(D, H * O), full(1, H * O),
            full(D, H * O), full(1, H * O),
            full(H, O),
        ],
        out_specs=[
            pl.BlockSpec((BQF, D), lambda i, j: (i, 0)),
            pl.BlockSpec((BQF, H * O), lambda i, j: (i, 0)),
            pl.BlockSpec((BQF, 16), lambda i, j: (i, 0)),
        ],
        out_shape=[
            jax.ShapeDtypeStruct((NPAD, D), _f32),
            jax.ShapeDtypeStruct((NPAD, H * O), _f32),
            jax.ShapeDtypeStruct((NPAD, 16), _f32),
        ],
        scratch_shapes=[
            pltpu.VMEM((BQF, 128), _f32),
            pltpu.VMEM((BQF, 128), _f32),
            pltpu.VMEM((BQF, DZ), _f32),
        ],
        compiler_params=pltpu.CompilerParams(
            dimension_semantics=("parallel", "arbitrary")),
    )(q, k, v, embp, proj_W, proj_b.reshape(1, -1), lin_W,
      lin_b.reshape(1, -1), ln_g.reshape(1, -1), ln_b.reshape(1, -1),
      Wt, bt.reshape(1, -1), Wi, bi.reshape(1, -1), Wj, bj.reshape(1, -1),
      attn)


def _tables_body(e1_ref, skW_ref, skb_ref,
                 Wt_ref, bt_ref, Wi_ref, bi_ref, Wj_ref, bj_ref, attn_ref,
                 s0_out, xtn_out, aij_out):
    e1 = e1_ref[...]
    s0_out[...] = jnp.dot(e1, skW_ref[...],
                          preferred_element_type=_f32) + skb_ref[...]
    xtn_out[...] = jnp.dot(e1, Wt_ref[...],
                           preferred_element_type=_f32) + bt_ref[...]
    attnm = attn_ref[...]
    wA = jnp.concatenate([_head_fold(Wi_ref[...], attnm),
                          _head_fold(Wj_ref[...], attnm)], axis=1)
    cA = jnp.concatenate([_bias_fold(bi_ref[...], attnm),
                          _bias_fold(bj_ref[...], attnm)], axis=1)
    aij8 = jnp.dot(e1, wA, preferred_element_type=_f32) + cA
    aij_out[...] = jnp.concatenate([aij8, jnp.zeros((256, 8), _f32)], axis=1)


def _tables_call(e1, skW, skb, Wt, bt, Wi, bi, Wj, bj, attn):
    BQ = 256
    grid = (NPAD // BQ,)
    full = lambda r, c: pl.BlockSpec((r, c), lambda i: (0, 0))
    blk = lambda c: pl.BlockSpec((BQ, c), lambda i: (i, 0))
    return pl.pallas_call(
        _tables_body,
        grid=grid,
        in_specs=[
            blk(D),
            full(O, O), full(1, O),
            full(D, H * O), full(1, H * O),
            full(D, H * O), full(1, H * O),
            full(D, H * O), full(1, H * O),
            full(H, O),
        ],
        out_specs=[blk(D), blk(H * O), blk(16)],
        out_shape=[
            jax.ShapeDtypeStruct((NPAD, D), _f32),
            jax.ShapeDtypeStruct((NPAD, H * O), _f32),
            jax.ShapeDtypeStruct((NPAD, 16), _f32),
        ],
    )(e1, skW, skb.reshape(1, -1), Wt, bt.reshape(1, -1),
      Wi, bi.reshape(1, -1), Wj, bj.reshape(1, -1), attn)


def _final_body(e2_ref, s0_ref, skW_ref, skb_ref, g_ref, b_ref,
                out_ref):
    e2 = e2_ref[...]
    s1 = jnp.dot(e2, skW_ref[...], preferred_element_type=_f32) + skb_ref[...]
    out = e2 + (s0_ref[...] + s1) * 0.5
    mu = jnp.mean(out, axis=1, keepdims=True)
    var = jnp.mean((out - mu) * (out - mu), axis=1, keepdims=True)
    out_ref[...] = (out - mu) * lax.rsqrt(var + 1e-5) * g_ref[...] + b_ref[...]


def _final_call(e2, s0, skW, skb, nf_g, nf_b):
    BQ = 256
    grid = (NPAD // BQ,)
    full = lambda r, c: pl.BlockSpec((r, c), lambda i: (0, 0))
    blk = lambda c: pl.BlockSpec((BQ, c), lambda i: (i, 0))
    return pl.pallas_call(
        _final_body,
        grid=grid,
        in_specs=[blk(D), blk(D), full(O, O), full(1, O),
                  full(1, O), full(1, O)],
        out_specs=blk(D),
        out_shape=jax.ShapeDtypeStruct((NPAD, D), _f32),
    )(e2, s0, skW, skb.reshape(1, -1), nf_g.reshape(1, -1),
      nf_b.reshape(1, -1))


NG = CH // 16


def _mesh():
    return plsc.VectorSubcoreMesh(core_axis_name="c", subcore_axis_name="s",
                                  num_cores=NC, num_subcores=NS)


def _scalar_lane(vec, lane, iota):
    return jnp.max(jnp.where(iota == lane, vec, -3e38))


def _edge_stats_body(src_hbm, dst_hbm, ea_hbm, aijp_hbm,
                     m_hbm, z_hbm,
                     srcv, dstv, eav, pidxd, pidxs, gdp, gsp, row16, sem):
    cid = lax.axis_index("c")
    sid = lax.axis_index("s")
    wid = cid * NS + sid
    iota = lax.iota(_i32, 16)

    def chunk_body(c, carry):
        accs = list(carry)
        base = wid * EPW + c * CH
        pltpu.sync_copy(src_hbm.at[pl.ds(base, CH)], srcv)
        pltpu.sync_copy(dst_hbm.at[pl.ds(base, CH)], dstv)
        pltpu.sync_copy(ea_hbm.at[pl.ds(base, CH)], eav)
        for g in range(NG):
            sl = pl.ds(g * 16, 16)
            pidxd[sl] = lax.shift_right_logical(dstv[sl], 3)
            pidxs[sl] = lax.shift_right_logical(srcv[sl], 3)
        cp1 = pltpu.async_copy(aijp_hbm.at[pidxd], gdp, sem)
        cp2 = pltpu.async_copy(aijp_hbm.at[pidxs], gsp, sem)
        cp1.wait()
        cp2.wait()
        for g in range(NG):
            sl = pl.ds(g * 16, 16)
            rows = iota + g * 16
            offd = jnp.bitwise_and(dstv[sl], 7) * 16
            offs = jnp.bitwise_and(srcv[sl], 7) * 16 + 4
            ea16 = eav[sl]
            for h in range(H):
                ga = plsc.load_gather(gdp, [rows, offd + h])
                gj = plsc.load_gather(gsp, [rows, offs + h])
                sh = (ga + gj) * ea16
                m_a, z_a = accs[2 * h], accs[2 * h + 1]
                m_n = jnp.maximum(m_a, sh)
                accs[2 * h] = m_n
                accs[2 * h + 1] = (z_a * jnp.exp(m_a - m_n)
                                   + jnp.exp(sh - m_n))
        return tuple(accs)

    init = []
    for _ in range(H):
        init += [jnp.full((16,), -3e38, _f32), jnp.zeros((16,), _f32)]
    accs = lax.fori_loop(0, NCHUNK, chunk_body, tuple(init))

    mvec = jnp.full((16,), -3e38, _f32)
    zvec = jnp.zeros((16,), _f32)
    for h in range(H):
        mh = jnp.max(accs[2 * h])
        zh = jnp.sum(accs[2 * h + 1] * jnp.exp(accs[2 * h] - mh))
        mvec = jnp.where(iota == h, mh, mvec)
        zvec = jnp.where(iota == h, zh, zvec)
    row16[...] = mvec
    pltpu.sync_copy(row16, m_hbm.at[wid])
    row16[...] = zvec
    pltpu.sync_copy(row16, z_hbm.at[wid])


def _edge_stats_call(src, dst, ea, aijp):
    kfn = pl.kernel(
        _edge_stats_body,
        out_type=[jax.ShapeDtypeStruct((NW, 16), _f32),
                  jax.ShapeDtypeStruct((NW, 16), _f32)],
        mesh=_mesh(),
        scratch_types=[
            pltpu.VMEM((CH,), _i32),
            pltpu.VMEM((CH,), _i32),
            pltpu.VMEM((CH,), _f32),
            pltpu.VMEM((CH,), _i32),
            pltpu.VMEM((CH,), _i32),
            pltpu.VMEM((CH, 128), _f32),
            pltpu.VMEM((CH, 128), _f32),
            pltpu.VMEM((16,), _f32),
            pltpu.SemaphoreType.DMA,
        ],
        compiler_params=pltpu.CompilerParams(needs_layout_passes=False),
    )
    return kfn(src, dst, ea, aijp)


def _edge_aggr_body(src_hbm, dst_hbm, ea_hbm, aijp_hbm, xtn_hbm,
                    m_hbm, z_hbm, zrows_hbm,
                    part_hbm,
                    srcv, dstv, eav, pidxd, pidxs, sctv, gdp, gsp, xrv,
                    moutv, mzv, abuf, acc_sh, sem):
    cid = lax.axis_index("c")
    sid = lax.axis_index("s")
    iota = lax.iota(_i32, 16)
    rowbase = cid * HALF

    pltpu.sync_copy(zrows_hbm, acc_sh.at[pl.ds(sid * RPS2, RPS2)])

    pltpu.sync_copy(m_hbm, mzv.at[pl.ds(0, NW)])
    pltpu.sync_copy(z_hbm, mzv.at[pl.ds(NW, NW)])

    def mfold(w, m_a):
        return jnp.maximum(m_a, plsc.load_gather(
            mzv, [jnp.full((16,), w, _i32), iota]))
    M = lax.fori_loop(0, NW, mfold, jnp.full((16,), -3e38, _f32))

    def zfold(w, z_a):
        fw = jnp.full((16,), w, _i32)
        mw = plsc.load_gather(mzv, [fw, iota])
        zw = plsc.load_gather(mzv, [fw + NW, iota])
        return z_a + zw * jnp.exp(mw - M)
    Z = lax.fori_loop(0, NW, zfold, jnp.zeros((16,), _f32))

    Mh = [_scalar_lane(M, h, iota) for h in range(H)]
    invZ = jnp.ones((16,), _f32) / jnp.where(iota < H, Z, 1.0)
    iZh = [jnp.max(jnp.where(iota == h, invZ, -3e38)) for h in range(H)]

    plsc.subcore_barrier()

    def chunk_body(c, _):
        base = sid * EPW2 + c * CH
        pltpu.sync_copy(src_hbm.at[pl.ds(base, CH)], srcv)
        pltpu.sync_copy(dst_hbm.at[pl.ds(base, CH)], dstv)
        pltpu.sync_copy(ea_hbm.at[pl.ds(base, CH)], eav)
        for g in range(NG):
            sl = pl.ds(g * 16, 16)
            d16 = dstv[sl]
            pidxd[sl] = lax.shift_right_logical(d16, 3)
            pidxs[sl] = lax.shift_right_logical(srcv[sl], 3)
            loc = d16 - rowbase
            ok = jnp.logical_and(loc >= 0, loc < HALF)
            sctv[sl] = jnp.where(ok, loc, HALF)
        cp1 = pltpu.async_copy(aijp_hbm.at[pidxd], gdp, sem)
        cp2 = pltpu.async_copy(aijp_hbm.at[pidxs], gsp, sem)
        cp3 = pltpu.async_copy(xtn_hbm.at[srcv], xrv, sem)
        cp1.wait()
        cp2.wait()
        for g in range(NG):
            sl = pl.ds(g * 16, 16)
            rows = iota + g * 16
            offd = jnp.bitwise_and(dstv[sl], 7) * 16
            offs = jnp.bitwise_and(srcv[sl], 7) * 16 + 4
            ea16 = eav[sl]
            for h in range(H):
                ga = plsc.load_gather(gdp, [rows, offd + h])
                gj = plsc.load_gather(gsp, [rows, offs + h])
                sh = (ga + gj) * ea16
                abuf[h, sl] = jnp.exp(sh - Mh[h]) * iZh[h]
        cp3.wait()

        def ebody(e, _e):
            fe = jnp.full((16,), e, _i32)
            acc = [None] * 8
            for h in range(H):
                w = plsc.load_gather(abuf, [jnp.full((16,), h, _i32), fe])
                for ct in range(8):
                    xv = plsc.load_gather(xrv, [fe, iota + (h * O + ct * 16)])
                    acc[ct] = w * xv if h == 0 else acc[ct] + w * xv
            for ct in range(8):
                plsc.store_scatter(moutv, [fe, iota + ct * 16], acc[ct])
            return 0

        lax.fori_loop(0, CH, ebody, 0)
        pltpu.sync_copy(moutv, acc_sh.at[sctv], add=True)
        return 0

    lax.fori_loop(0, NCHUNK2, chunk_body, 0)

    plsc.subcore_barrier()
    pltpu.sync_copy(acc_sh.at[pl.ds(sid * RPS2, RPS2)],
                    part_hbm.at[cid, pl.ds(sid * RPS2, RPS2)])


def _edge_aggr_call(src, dst, ea, aijp, xtn, m_p, z_p, zrows):
    kfn = pl.kernel(
        _edge_aggr_body,
        out_type=jax.ShapeDtypeStruct((NC, HALF_PAD, D), _f32),
        mesh=_mesh(),
        scratch_types=[
            pltpu.VMEM((CH,), _i32),
            pltpu.VMEM((CH,), _i32),
            pltpu.VMEM((CH,), _f32),
            pltpu.VMEM((CH,), _i32),
            pltpu.VMEM((CH,), _i32),
            pltpu.VMEM((CH,), _i32),
            pltpu.VMEM((CH, 128), _f32),
            pltpu.VMEM((CH, 128), _f32),
            pltpu.VMEM((CH, H * O), _f32),
            pltpu.VMEM((CH, D), _f32),
            pltpu.VMEM((2 * NW, 16), _f32),
            pltpu.VMEM((H, CH), _f32),
            pltpu.VMEM_SHARED((HALF_PAD, D), _f32),
            pltpu.SemaphoreType.DMA,
        ],
        compiler_params=pltpu.CompilerParams(needs_layout_passes=False),
    )
    return kfn(src, dst, ea, aijp, xtn, m_p, z_p, zrows)


def _edge_layer(src, dst, ea, aij, xtn, zrows):
    aijp = aij.reshape(NPAD // 8, 128)
    m_p, z_p = _edge_stats_call(src, dst, ea, aijp)
    part = _edge_aggr_call(src, dst, ea, aijp, xtn, m_p, z_p, zrows)
    return jnp.concatenate([part[0, :HALF], part[1, :HALF]], axis=0)


@jax.jit
def kernel(x, ids, edge_index, edge_attr, embed, qkv_W, qkv_b, proj_W,
           proj_b, lin_W, lin_b, ln_g, ln_b,
           Wi0, bi0, Wj0, bj0, Wt0, bt0, attn0, skW0, skb0,
           Wi1, bi1, Wj1, bj1, Wt1, bt1, attn1, skW1, skb1,
           nf_g, nf_b):
    Ei = jnp.take(embed, ids, axis=0)
    xp = jnp.pad(x, ((0, NPAD - N), (0, 0)))
    embp = jnp.pad(Ei, ((0, NPAD - N), (0, 0)))
    src = edge_index[0].astype(_i32)
    dst = edge_index[1].astype(_i32)
    zrows = jnp.zeros((RPS2, D), _f32)

    q, k, v = _qkv_call(xp, embp, qkv_W, qkv_b)
    emb, xtn0, aij0 = _flash_call(q, k, v, embp, proj_W, proj_b, lin_W, lin_b,
                                  ln_g, ln_b, Wt0, bt0, Wi0, bi0, Wj0, bj0,
                                  attn0)
    e1 = _edge_layer(src, dst, edge_attr, aij0, xtn0, zrows)
    s0, xtn1, aij1 = _tables_call(e1, skW0, skb0, Wt1, bt1,
                                  Wi1, bi1, Wj1, bj1, attn1)
    e2 = _edge_layer(src, dst, edge_attr, aij1, xtn1, zrows)
    outp = _final_call(e2, s0, skW1, skb1, nf_g, nf_b)
    return outp[:N]


# speedup vs baseline: 1.5771x; 1.2011x over previous
import functools
import jax
import jax.numpy as jnp
from jax import lax
from jax.experimental import pallas as pl
from jax.experimental.pallas import tpu as pltpu
from jax.experimental.pallas import tpu_sc as plsc

N = 10000
E = 320000
D = 128
XI = 64
H = 4
O = 128
DZ = D + XI
NPAD = 10240
SCALE = float(DZ) ** -0.5

NC = 2
NS = 16
NW = NC * NS
EPW = E // NW
CH = 80
NCHUNK = EPW // CH
ROWS_PER_SUB = NPAD // NS
HCOL = 64
ACCR = NPAD // 2
EPW2 = E // NS
NCHUNK2 = EPW2 // CH
RPS2 = ACCR // NS

_f32 = jnp.float32
_i32 = jnp.int32


def _qkv_body(x_ref, emb_ref, wq_ref, wk_ref, wv_ref, b_ref,
              q_out, k_out, v_out):
    x = x_ref[...]
    emb = emb_ref[...]
    b = b_ref[...]
    for wr, out, col in ((wq_ref, q_out, 0), (wk_ref, k_out, 1),
                         (wv_ref, v_out, 2)):
        w = wr[...]
        wsum = jnp.sum(w[:XI, :], axis=0, keepdims=True)
        out[...] = (x * wsum
                    + jnp.dot(emb, w[XI:, :], preferred_element_type=_f32)
                    + b[col:col + 1, :])


def _qkv_call(xp, embp, qkv_W, qkv_b):
    BQ = 512
    grid = (NPAD // BQ,)
    out_blk = pl.BlockSpec((BQ, DZ), lambda i: (i, 0))
    return pl.pallas_call(
        _qkv_body,
        grid=grid,
        in_specs=[
            pl.BlockSpec((BQ, 1), lambda i: (i, 0)),
            pl.BlockSpec((BQ, D), lambda i: (i, 0)),
            pl.BlockSpec((DZ, DZ), lambda i: (0, 0)),
            pl.BlockSpec((DZ, DZ), lambda i: (0, 0)),
            pl.BlockSpec((DZ, DZ), lambda i: (0, 0)),
            pl.BlockSpec((3, DZ), lambda i: (0, 0)),
        ],
        out_specs=[out_blk, out_blk, out_blk],
        out_shape=[jax.ShapeDtypeStruct((NPAD, DZ), _f32)] * 3,
    )(xp, embp, qkv_W[:, :DZ], qkv_W[:, DZ:2 * DZ], qkv_W[:, 2 * DZ:],
      qkv_b.reshape(3, DZ))


BQF = 256
BKF = 1024
NKF = NPAD // BKF


def _head_fold(Wm, attnm):
    cols = [jnp.sum(Wm[:, h * O:(h + 1) * O] * attnm[h:h + 1, :],
                    axis=1, keepdims=True) for h in range(H)]
    return jnp.concatenate(cols, axis=1)


def _bias_fold(bm, attnm):
    cols = [jnp.sum(bm[:, h * O:(h + 1) * O] * attnm[h:h + 1, :],
                    axis=1, keepdims=True) for h in range(H)]
    return jnp.concatenate(cols, axis=1)


def _flash_body(q_ref, k_ref, v_ref, ei_ref,
                projW_ref, projb_ref, linW_ref, linb_ref, lng_ref, lnb_ref,
                Wt_ref, bt_ref, Wi_ref, bi_ref, Wj_ref, bj_ref, attn_ref,
                emb_out, xtn_out, aij_out,
                m_scr, l_scr, acc_scr):
    j = pl.program_id(1)

    @pl.when(j == 0)
    def _():
        m_scr[...] = jnp.full((BQF, 128), -3e38, _f32)
        l_scr[...] = jnp.zeros((BQF, 128), _f32)
        acc_scr[...] = jnp.zeros((BQF, DZ), _f32)

    q = q_ref[...]
    k = k_ref[...]
    logits = lax.dot_general(q, k, (((1,), (1,)), ((), ())),
                             preferred_element_type=_f32) * SCALE
    col = j * BKF + lax.broadcasted_iota(_i32, (BQF, BKF), 1)
    logits = jnp.where(col < N, logits, -3e38)

    m_prev = m_scr[:, 0:1]
    l_prev = l_scr[:, 0:1]
    m_new = jnp.maximum(m_prev, jnp.max(logits, axis=1, keepdims=True))
    alpha = jnp.exp(m_prev - m_new)
    p = jnp.exp(logits - m_new)
    l_new = l_prev * alpha + jnp.sum(p, axis=1, keepdims=True)
    acc_new = acc_scr[...] * alpha + jnp.dot(p, v_ref[...],
                                             preferred_element_type=_f32)
    m_scr[...] = jnp.broadcast_to(m_new, (BQF, 128))
    l_scr[...] = jnp.broadcast_to(l_new, (BQF, 128))
    acc_scr[...] = acc_new

    @pl.when(j == NKF - 1)
    def _():
        o = acc_new / l_new
        o2 = jnp.dot(o, projW_ref[...], preferred_element_type=_f32) \
            + projb_ref[...]
        hh = jnp.dot(o2, linW_ref[...], preferred_element_type=_f32) \
            + linb_ref[...]
        mu = jnp.mean(hh, axis=1, keepdims=True)
        var = jnp.mean((hh - mu) * (hh - mu), axis=1, keepdims=True)
        hln = (hh - mu) * lax.rsqrt(var + 1e-5) * lng_ref[...] + lnb_ref[...]
        hs = hln * jax.nn.sigmoid(hln)
        embv = hs + ei_ref[...]
        emb_out[...] = embv
        xtn_out[...] = jnp.dot(embv, Wt_ref[...],
                               preferred_element_type=_f32) + bt_ref[...]
        attnm = attn_ref[...]
        wA = jnp.concatenate([_head_fold(Wi_ref[...], attnm),
                              _head_fold(Wj_ref[...], attnm)], axis=1)
        cA = jnp.concatenate([_bias_fold(bi_ref[...], attnm),
                              _bias_fold(bj_ref[...], attnm)], axis=1)
        aij8 = jnp.dot(embv, wA, preferred_element_type=_f32) + cA
        aij_out[...] = jnp.concatenate(
            [aij8, jnp.zeros((BQF, 8), _f32)], axis=1)


def _flash_call(q, k, v, embp, proj_W, proj_b, lin_W, lin_b, ln_g, ln_b,
                Wt, bt, Wi, bi, Wj, bj, attn):
    grid = (NPAD // BQF, NKF)
    full = lambda r, c: pl.BlockSpec((r, c), lambda i, j: (0, 0))
    return pl.pallas_call(
        _flash_body,
        grid=grid,
        in_specs=[
            pl.BlockSpec((BQF, DZ), lambda i, j: (i, 0)),
            pl.BlockSpec((BKF, DZ), lambda i, j: (j, 0)),
            pl.BlockSpec((BKF, DZ), lambda i, j: (j, 0)),
            pl.BlockSpec((BQF, D), lambda i, j: (i, 0)),
            full(DZ, DZ), full(1, DZ), full(DZ, D), full(1, D),
            full(1, D), full(1, D),
            full(D, H * O), full(1, H * O),
            full(D, H * O), full(1, H * O),
            full(D, H * O), full(1, H * O),
            full(H, O),
        ],
        out_specs=[
            pl.BlockSpec((BQF, D), lambda i, j: (i, 0)),
            pl.BlockSpec((BQF, H * O), lambda i, j: (i, 0)),
            pl.BlockSpec((BQF, 16), lambda i, j: (i, 0)),
        ],
        out_shape=[
            jax.ShapeDtypeStruct((NPAD, D), _f32),
            jax.ShapeDtypeStruct((NPAD, H * O), _f32),
            jax.ShapeDtypeStruct((NPAD, 16), _f32),
        ],
        scratch_shapes=[
            pltpu.VMEM((BQF, 128), _f32),
            pltpu.VMEM((BQF, 128), _f32),
            pltpu.VMEM((BQF, DZ), _f32),
        ],
        compiler_params=pltpu.CompilerParams(
            dimension_semantics=("parallel", "arbitrary")),
    )(q, k, v, embp, proj_W, proj_b.reshape(1, -1), lin_W,
      lin_b.reshape(1, -1), ln_g.reshape(1, -1), ln_b.reshape(1, -1),
      Wt, bt.reshape(1, -1), Wi, bi.reshape(1, -1), Wj, bj.reshape(1, -1),
      attn)


def _tables_body(e1_ref, skW_ref, skb_ref,
                 Wt_ref, bt_ref, Wi_ref, bi_ref, Wj_ref, bj_ref, attn_ref,
                 s0_out, xtn_out, aij_out):
    e1 = e1_ref[...]
    s0_out[...] = jnp.dot(e1, skW_ref[...],
                          preferred_element_type=_f32) + skb_ref[...]
    xtn_out[...] = jnp.dot(e1, Wt_ref[...],
                           preferred_element_type=_f32) + bt_ref[...]
    attnm = attn_ref[...]
    wA = jnp.concatenate([_head_fold(Wi_ref[...], attnm),
                          _head_fold(Wj_ref[...], attnm)], axis=1)
    cA = jnp.concatenate([_bias_fold(bi_ref[...], attnm),
                          _bias_fold(bj_ref[...], attnm)], axis=1)
    aij8 = jnp.dot(e1, wA, preferred_element_type=_f32) + cA
    aij_out[...] = jnp.concatenate([aij8, jnp.zeros((256, 8), _f32)], axis=1)


def _tables_call(e1, skW, skb, Wt, bt, Wi, bi, Wj, bj, attn):
    BQ = 256
    grid = (NPAD // BQ,)
    full = lambda r, c: pl.BlockSpec((r, c), lambda i: (0, 0))
    blk = lambda c: pl.BlockSpec((BQ, c), lambda i: (i, 0))
    return pl.pallas_call(
        _tables_body,
        grid=grid,
        in_specs=[
            blk(D),
            full(O, O), full(1, O),
            full(D, H * O), full(1, H * O),
            full(D, H * O), full(1, H * O),
            full(D, H * O), full(1, H * O),
            full(H, O),
        ],
        out_specs=[blk(D), blk(H * O), blk(16)],
        out_shape=[
            jax.ShapeDtypeStruct((NPAD, D), _f32),
            jax.ShapeDtypeStruct((NPAD, H * O), _f32),
            jax.ShapeDtypeStruct((NPAD, 16), _f32),
        ],
    )(e1, skW, skb.reshape(1, -1), Wt, bt.reshape(1, -1),
      Wi, bi.reshape(1, -1), Wj, bj.reshape(1, -1), attn)


def _final_body(e2_ref, s0_ref, skW_ref, skb_ref, g_ref, b_ref,
                out_ref):
    e2 = e2_ref[...]
    s1 = jnp.dot(e2, skW_ref[...], preferred_element_type=_f32) + skb_ref[...]
    out = e2 + (s0_ref[...] + s1) * 0.5
    mu = jnp.mean(out, axis=1, keepdims=True)
    var = jnp.mean((out - mu) * (out - mu), axis=1, keepdims=True)
    out_ref[...] = (out - mu) * lax.rsqrt(var + 1e-5) * g_ref[...] + b_ref[...]


def _final_call(e2, s0, skW, skb, nf_g, nf_b):
    BQ = 256
    grid = (NPAD // BQ,)
    full = lambda r, c: pl.BlockSpec((r, c), lambda i: (0, 0))
    blk = lambda c: pl.BlockSpec((BQ, c), lambda i: (i, 0))
    return pl.pallas_call(
        _final_body,
        grid=grid,
        in_specs=[blk(D), blk(D), full(O, O), full(1, O),
                  full(1, O), full(1, O)],
        out_specs=blk(D),
        out_shape=jax.ShapeDtypeStruct((NPAD, D), _f32),
    )(e2, s0, skW, skb.reshape(1, -1), nf_g.reshape(1, -1),
      nf_b.reshape(1, -1))


NG = CH // 16


def _mesh():
    return plsc.VectorSubcoreMesh(core_axis_name="c", subcore_axis_name="s",
                                  num_cores=NC, num_subcores=NS)


def _scalar_lane(vec, lane, iota):
    return jnp.max(jnp.where(iota == lane, vec, -3e38))


def _edge_stats_body(src_hbm, dst_hbm, ea_hbm, aijp_hbm,
                     m_hbm, z_hbm,
                     srcv, dstv, eav, pidxd, pidxs, gdp, gsp, row16, sem):
    cid = lax.axis_index("c")
    sid = lax.axis_index("s")
    wid = cid * NS + sid
    iota = lax.iota(_i32, 16)

    def chunk_body(c, carry):
        accs = list(carry)
        base = wid * EPW + c * CH
        pltpu.sync_copy(src_hbm.at[pl.ds(base, CH)], srcv)
        pltpu.sync_copy(dst_hbm.at[pl.ds(base, CH)], dstv)
        pltpu.sync_copy(ea_hbm.at[pl.ds(base, CH)], eav)
        for g in range(NG):
            sl = pl.ds(g * 16, 16)
            pidxd[sl] = lax.shift_right_logical(dstv[sl], 3)
            pidxs[sl] = lax.shift_right_logical(srcv[sl], 3)
        cp1 = pltpu.async_copy(aijp_hbm.at[pidxd], gdp, sem)
        cp2 = pltpu.async_copy(aijp_hbm.at[pidxs], gsp, sem)
        cp1.wait()
        cp2.wait()
        for g in range(NG):
            sl = pl.ds(g * 16, 16)
            rows = iota + g * 16
            offd = jnp.bitwise_and(dstv[sl], 7) * 16
            offs = jnp.bitwise_and(srcv[sl], 7) * 16 + 4
            ea16 = eav[sl]
            for h in range(H):
                ga = plsc.load_gather(gdp, [rows, offd + h])
                gj = plsc.load_gather(gsp, [rows, offs + h])
                sh = (ga + gj) * ea16
                m_a, z_a = accs[2 * h], accs[2 * h + 1]
                m_n = jnp.maximum(m_a, sh)
                accs[2 * h] = m_n
                accs[2 * h + 1] = (z_a * jnp.exp(m_a - m_n)
                                   + jnp.exp(sh - m_n))
        return tuple(accs)

    init = []
    for _ in range(H):
        init += [jnp.full((16,), -3e38, _f32), jnp.zeros((16,), _f32)]
    accs = lax.fori_loop(0, NCHUNK, chunk_body, tuple(init))

    mvec = jnp.full((16,), -3e38, _f32)
    zvec = jnp.zeros((16,), _f32)
    for h in range(H):
        mh = jnp.max(accs[2 * h])
        zh = jnp.sum(accs[2 * h + 1] * jnp.exp(accs[2 * h] - mh))
        mvec = jnp.where(iota == h, mh, mvec)
        zvec = jnp.where(iota == h, zh, zvec)
    row16[...] = mvec
    pltpu.sync_copy(row16, m_hbm.at[wid])
    row16[...] = zvec
    pltpu.sync_copy(row16, z_hbm.at[wid])


def _edge_stats_call(src, dst, ea, aijp):
    kfn = pl.kernel(
        _edge_stats_body,
        out_type=[jax.ShapeDtypeStruct((NW, 16), _f32),
                  jax.ShapeDtypeStruct((NW, 16), _f32)],
        mesh=_mesh(),
        scratch_types=[
            pltpu.VMEM((CH,), _i32),
            pltpu.VMEM((CH,), _i32),
            pltpu.VMEM((CH,), _f32),
            pltpu.VMEM((CH,), _i32),
            pltpu.VMEM((CH,), _i32),
            pltpu.VMEM((CH, 128), _f32),
            pltpu.VMEM((CH, 128), _f32),
            pltpu.VMEM((16,), _f32),
            pltpu.SemaphoreType.DMA,
        ],
        compiler_params=pltpu.CompilerParams(needs_layout_passes=False),
    )
    return kfn(src, dst, ea, aijp)


def _edge_aggr_body(src_hbm, dst_hbm, ea_hbm, aijp_hbm, xtn_hbm,
                    m_hbm, z_hbm, zrows_hbm,
                    part_hbm,
                    srcv, dstv, eav, pidxd, pidxs, sctv, pcv, gdp, gsp, xrv,
                    moutv, mzv, abuf, acc_sh, sem):
    cid = lax.axis_index("c")
    sid = lax.axis_index("s")
    iota = lax.iota(_i32, 16)
    rowoff = cid * NPAD

    pltpu.sync_copy(zrows_hbm, acc_sh.at[pl.ds(sid * RPS2, RPS2)])

    pltpu.sync_copy(m_hbm, mzv.at[pl.ds(0, NW)])
    pltpu.sync_copy(z_hbm, mzv.at[pl.ds(NW, NW)])

    def mfold(w, m_a):
        return jnp.maximum(m_a, plsc.load_gather(
            mzv, [jnp.full((16,), w, _i32), iota]))
    M = lax.fori_loop(0, NW, mfold, jnp.full((16,), -3e38, _f32))

    def zfold(w, z_a):
        fw = jnp.full((16,), w, _i32)
        mw = plsc.load_gather(mzv, [fw, iota])
        zw = plsc.load_gather(mzv, [fw + NW, iota])
        return z_a + zw * jnp.exp(mw - M)
    Z = lax.fori_loop(0, NW, zfold, jnp.zeros((16,), _f32))

    Mh = [_scalar_lane(M, h, iota) for h in range(H)]
    invZ = jnp.ones((16,), _f32) / jnp.where(iota < H, Z, 1.0)
    iZh = [jnp.max(jnp.where(iota == h, invZ, -3e38)) for h in range(H)]

    plsc.subcore_barrier()

    zv = jnp.zeros((16,), _f32)

    def chunk_body(c, _):
        base = sid * EPW2 + c * CH
        pltpu.sync_copy(src_hbm.at[pl.ds(base, CH)], srcv)
        pltpu.sync_copy(dst_hbm.at[pl.ds(base, CH)], dstv)
        pltpu.sync_copy(ea_hbm.at[pl.ds(base, CH)], eav)
        for g in range(NG):
            sl = pl.ds(g * 16, 16)
            d16 = dstv[sl]
            s16 = srcv[sl]
            pidxd[sl] = lax.shift_right_logical(d16, 3)
            pidxs[sl] = lax.shift_right_logical(s16, 3)
            sctv[sl] = lax.shift_right_logical(d16, 1)
            pcv[0, sl] = jnp.bitwise_and(d16, 1) * HCOL
            srcv[sl] = s16 + rowoff
        cp1 = pltpu.async_copy(aijp_hbm.at[pidxd], gdp, sem)
        cp2 = pltpu.async_copy(aijp_hbm.at[pidxs], gsp, sem)
        cp3 = pltpu.async_copy(xtn_hbm.at[srcv], xrv, sem)
        cp1.wait()
        cp2.wait()
        for g in range(NG):
            sl = pl.ds(g * 16, 16)
            rows = iota + g * 16
            offd = jnp.bitwise_and(dstv[sl], 7) * 16
            offs = jnp.bitwise_and(srcv[sl], 7) * 16 + 4
            ea16 = eav[sl]
            for h in range(H):
                ga = plsc.load_gather(gdp, [rows, offd + h])
                gj = plsc.load_gather(gsp, [rows, offs + h])
                sh = (ga + gj) * ea16
                abuf[h, sl] = jnp.exp(sh - Mh[h]) * iZh[h]
        cp3.wait()

        def ebody(e, _e):
            fe = jnp.full((16,), e, _i32)
            cw = plsc.load_gather(pcv, [jnp.zeros((16,), _i32), fe])
            acc = [None] * 4
            for h in range(H):
                w = plsc.load_gather(abuf, [jnp.full((16,), h, _i32), fe])
                for ct in range(4):
                    xv = plsc.load_gather(xrv,
                                          [fe, iota + (h * HCOL + ct * 16)])
                    acc[ct] = w * xv if h == 0 else acc[ct] + w * xv
            for ct in range(4):
                plsc.store_scatter(moutv, [fe, iota + ct * 16 + cw], acc[ct])
                plsc.store_scatter(moutv, [fe, iota + ct * 16 + (HCOL - cw)],
                                   zv)
            return 0

        lax.fori_loop(0, CH, ebody, 0)
        pltpu.sync_copy(moutv, acc_sh.at[sctv], add=True)
        return 0

    lax.fori_loop(0, NCHUNK2, chunk_body, 0)

    plsc.subcore_barrier()
    pltpu.sync_copy(acc_sh.at[pl.ds(sid * RPS2, RPS2)],
                    part_hbm.at[cid, pl.ds(sid * RPS2, RPS2)])


def _edge_aggr_call(src, dst, ea, aijp, xtn, m_p, z_p, zrows):
    kfn = pl.kernel(
        _edge_aggr_body,
        out_type=jax.ShapeDtypeStruct((NC, ACCR, D), _f32),
        mesh=_mesh(),
        scratch_types=[
            pltpu.VMEM((CH,), _i32),
            pltpu.VMEM((CH,), _i32),
            pltpu.VMEM((CH,), _f32),
            pltpu.VMEM((CH,), _i32),
            pltpu.VMEM((CH,), _i32),
            pltpu.VMEM((CH,), _i32),
            pltpu.VMEM((1, CH), _i32),
            pltpu.VMEM((CH, 128), _f32),
            pltpu.VMEM((CH, 128), _f32),
            pltpu.VMEM((CH, H * HCOL), _f32),
            pltpu.VMEM((CH, D), _f32),
            pltpu.VMEM((2 * NW, 16), _f32),
            pltpu.VMEM((H, CH), _f32),
            pltpu.VMEM_SHARED((ACCR, D), _f32),
            pltpu.SemaphoreType.DMA,
        ],
        compiler_params=pltpu.CompilerParams(needs_layout_passes=False),
    )
    return kfn(src, dst, ea, aijp, xtn, m_p, z_p, zrows)


def _edge_layer(src, dst, ea, aij, xtn, zrows):
    aijp = aij.reshape(NPAD // 8, 128)
    xtn2 = jnp.concatenate([xtn[:, :H * HCOL], xtn[:, H * HCOL:]], axis=0)
    m_p, z_p = _edge_stats_call(src, dst, ea, aijp)
    part = _edge_aggr_call(src, dst, ea, aijp, xtn2, m_p, z_p, zrows)
    return jnp.concatenate([part[0].reshape(NPAD, HCOL),
                            part[1].reshape(NPAD, HCOL)], axis=1)


@jax.jit
def kernel(x, ids, edge_index, edge_attr, embed, qkv_W, qkv_b, proj_W,
           proj_b, lin_W, lin_b, ln_g, ln_b,
           Wi0, bi0, Wj0, bj0, Wt0, bt0, attn0, skW0, skb0,
           Wi1, bi1, Wj1, bj1, Wt1, bt1, attn1, skW1, skb1,
           nf_g, nf_b):
    Ei = jnp.take(embed, ids, axis=0)
    xp = jnp.pad(x, ((0, NPAD - N), (0, 0)))
    embp = jnp.pad(Ei, ((0, NPAD - N), (0, 0)))
    src = edge_index[0].astype(_i32)
    dst = edge_index[1].astype(_i32)
    zrows = jnp.zeros((RPS2, D), _f32)

    perm = jnp.asarray([h * O + hi * HCOL + j
                        for hi in range(2) for h in range(H)
                        for j in range(HCOL)], dtype=_i32)
    Wt0p, bt0p = Wt0[:, perm], bt0[perm]
    Wt1p, bt1p = Wt1[:, perm], bt1[perm]

    q, k, v = _qkv_call(xp, embp, qkv_W, qkv_b)
    emb, xtn0, aij0 = _flash_call(q, k, v, embp, proj_W, proj_b, lin_W, lin_b,
                                  ln_g, ln_b, Wt0p, bt0p, Wi0, bi0, Wj0, bj0,
                                  attn0)
    e1 = _edge_layer(src, dst, edge_attr, aij0, xtn0, zrows)
    s0, xtn1, aij1 = _tables_call(e1, skW0, skb0, Wt1p, bt1p,
                                  Wi1, bi1, Wj1, bj1, attn1)
    e2 = _edge_layer(src, dst, edge_attr, aij1, xtn1, zrows)
    outp = _final_call(e2, s0, skW1, skb1, nf_g, nf_b)
    return outp[:N]


# speedup vs baseline: 2.3489x; 1.4894x over previous
import functools
import jax
import jax.numpy as jnp
from jax import lax
from jax.experimental import pallas as pl
from jax.experimental.pallas import tpu as pltpu
from jax.experimental.pallas import tpu_sc as plsc

N = 10000
E = 320000
D = 128
XI = 64
H = 4
O = 128
DZ = D + XI
NPAD = 10240
SCALE = float(DZ) ** -0.5

NC = 2
NS = 16
NW = NC * NS
EPW = E // NW
CH = 80
NCHUNK = EPW // CH
ROWS_PER_SUB = NPAD // NS
HCOL = 64
ACCR = NPAD // 2
EPW2 = E // NS
NCHUNK2 = EPW2 // CH
RPS2 = ACCR // NS

_f32 = jnp.float32
_i32 = jnp.int32


def _qkv_body(x_ref, emb_ref, wq_ref, wk_ref, wv_ref, b_ref,
              q_out, k_out, v_out):
    x = x_ref[...]
    emb = emb_ref[...]
    b = b_ref[...]
    for wr, out, col in ((wq_ref, q_out, 0), (wk_ref, k_out, 1),
                         (wv_ref, v_out, 2)):
        w = wr[...]
        wsum = jnp.sum(w[:XI, :], axis=0, keepdims=True)
        out[...] = (x * wsum
                    + jnp.dot(emb, w[XI:, :], preferred_element_type=_f32)
                    + b[col:col + 1, :])


def _qkv_call(xp, embp, qkv_W, qkv_b):
    BQ = 512
    grid = (NPAD // BQ,)
    out_blk = pl.BlockSpec((BQ, DZ), lambda i: (i, 0))
    return pl.pallas_call(
        _qkv_body,
        grid=grid,
        in_specs=[
            pl.BlockSpec((BQ, 1), lambda i: (i, 0)),
            pl.BlockSpec((BQ, D), lambda i: (i, 0)),
            pl.BlockSpec((DZ, DZ), lambda i: (0, 0)),
            pl.BlockSpec((DZ, DZ), lambda i: (0, 0)),
            pl.BlockSpec((DZ, DZ), lambda i: (0, 0)),
            pl.BlockSpec((3, DZ), lambda i: (0, 0)),
        ],
        out_specs=[out_blk, out_blk, out_blk],
        out_shape=[jax.ShapeDtypeStruct((NPAD, DZ), _f32)] * 3,
    )(xp, embp, qkv_W[:, :DZ], qkv_W[:, DZ:2 * DZ], qkv_W[:, 2 * DZ:],
      qkv_b.reshape(3, DZ))


BQF = 256
BKF = 1024
NKF = NPAD // BKF


def _head_fold(Wm, attnm):
    cols = [jnp.sum(Wm[:, h * O:(h + 1) * O] * attnm[h:h + 1, :],
                    axis=1, keepdims=True) for h in range(H)]
    return jnp.concatenate(cols, axis=1)


def _bias_fold(bm, attnm):
    cols = [jnp.sum(bm[:, h * O:(h + 1) * O] * attnm[h:h + 1, :],
                    axis=1, keepdims=True) for h in range(H)]
    return jnp.concatenate(cols, axis=1)


def _flash_body(q_ref, k_ref, v_ref, ei_ref,
                projW_ref, projb_ref, linW_ref, linb_ref, lng_ref, lnb_ref,
                Wt_ref, bt_ref, Wi_ref, bi_ref, Wj_ref, bj_ref, attn_ref,
                emb_out, xtn_out, aij_out,
                m_scr, l_scr, acc_scr):
    j = pl.program_id(1)

    @pl.when(j == 0)
    def _():
        m_scr[...] = jnp.full((BQF, 128), -3e38, _f32)
        l_scr[...] = jnp.zeros((BQF, 128), _f32)
        acc_scr[...] = jnp.zeros((BQF, DZ), _f32)

    q = q_ref[...]
    k = k_ref[...]
    logits = lax.dot_general(q, k, (((1,), (1,)), ((), ())),
                             preferred_element_type=_f32) * SCALE
    col = j * BKF + lax.broadcasted_iota(_i32, (BQF, BKF), 1)
    logits = jnp.where(col < N, logits, -3e38)

    m_prev = m_scr[:, 0:1]
    l_prev = l_scr[:, 0:1]
    m_new = jnp.maximum(m_prev, jnp.max(logits, axis=1, keepdims=True))
    alpha = jnp.exp(m_prev - m_new)
    p = jnp.exp(logits - m_new)
    l_new = l_prev * alpha + jnp.sum(p, axis=1, keepdims=True)
    acc_new = acc_scr[...] * alpha + jnp.dot(p, v_ref[...],
                                             preferred_element_type=_f32)
    m_scr[...] = jnp.broadcast_to(m_new, (BQF, 128))
    l_scr[...] = jnp.broadcast_to(l_new, (BQF, 128))
    acc_scr[...] = acc_new

    @pl.when(j == NKF - 1)
    def _():
        o = acc_new / l_new
        o2 = jnp.dot(o, projW_ref[...], preferred_element_type=_f32) \
            + projb_ref[...]
        hh = jnp.dot(o2, linW_ref[...], preferred_element_type=_f32) \
            + linb_ref[...]
        mu = jnp.mean(hh, axis=1, keepdims=True)
        var = jnp.mean((hh - mu) * (hh - mu), axis=1, keepdims=True)
        hln = (hh - mu) * lax.rsqrt(var + 1e-5) * lng_ref[...] + lnb_ref[...]
        hs = hln * jax.nn.sigmoid(hln)
        embv = hs + ei_ref[...]
        emb_out[...] = embv
        xtn_out[...] = jnp.dot(embv, Wt_ref[...],
                               preferred_element_type=_f32) + bt_ref[...]
        attnm = attn_ref[...]
        wA = jnp.concatenate([_head_fold(Wi_ref[...], attnm),
                              _head_fold(Wj_ref[...], attnm)], axis=1)
        cA = jnp.concatenate([_bias_fold(bi_ref[...], attnm),
                              _bias_fold(bj_ref[...], attnm)], axis=1)
        aij8 = jnp.dot(embv, wA, preferred_element_type=_f32) + cA
        aij_out[...] = jnp.concatenate(
            [aij8, jnp.zeros((BQF, 8), _f32)], axis=1)


def _flash_call(q, k, v, embp, proj_W, proj_b, lin_W, lin_b, ln_g, ln_b,
                Wt, bt, Wi, bi, Wj, bj, attn):
    grid = (NPAD // BQF, NKF)
    full = lambda r, c: pl.BlockSpec((r, c), lambda i, j: (0, 0))
    return pl.pallas_call(
        _flash_body,
        grid=grid,
        in_specs=[
            pl.BlockSpec((BQF, DZ), lambda i, j: (i, 0)),
            pl.BlockSpec((BKF, DZ), lambda i, j: (j, 0)),
            pl.BlockSpec((BKF, DZ), lambda i, j: (j, 0)),
            pl.BlockSpec((BQF, D), lambda i, j: (i, 0)),
            full(DZ, DZ), full(1, DZ), full(DZ, D), full(1, D),
            full(1, D), full(1, D),
            full(D, H * O), full(1, H * O),
            full(D, H * O), full(1, H * O),
            full(D, H * O), full(1, H * O),
            full(H, O),
        ],
        out_specs=[
            pl.BlockSpec((BQF, D), lambda i, j: (i, 0)),
            pl.BlockSpec((BQF, H * O), lambda i, j: (i, 0)),
            pl.BlockSpec((BQF, 16), lambda i, j: (i, 0)),
        ],
        out_shape=[
            jax.ShapeDtypeStruct((NPAD, D), _f32),
            jax.ShapeDtypeStruct((NPAD, H * O), _f32),
            jax.ShapeDtypeStruct((NPAD, 16), _f32),
        ],
        scratch_shapes=[
            pltpu.VMEM((BQF, 128), _f32),
            pltpu.VMEM((BQF, 128), _f32),
            pltpu.VMEM((BQF, DZ), _f32),
        ],
        compiler_params=pltpu.CompilerParams(
            dimension_semantics=("parallel", "arbitrary")),
    )(q, k, v, embp, proj_W, proj_b.reshape(1, -1), lin_W,
      lin_b.reshape(1, -1), ln_g.reshape(1, -1), ln_b.reshape(1, -1),
      Wt, bt.reshape(1, -1), Wi, bi.reshape(1, -1), Wj, bj.reshape(1, -1),
      attn)


def _tables_body(e1_ref, skW_ref, skb_ref,
                 Wt_ref, bt_ref, Wi_ref, bi_ref, Wj_ref, bj_ref, attn_ref,
                 s0_out, xtn_out, aij_out):
    e1 = e1_ref[...]
    s0_out[...] = jnp.dot(e1, skW_ref[...],
                          preferred_element_type=_f32) + skb_ref[...]
    xtn_out[...] = jnp.dot(e1, Wt_ref[...],
                           preferred_element_type=_f32) + bt_ref[...]
    attnm = attn_ref[...]
    wA = jnp.concatenate([_head_fold(Wi_ref[...], attnm),
                          _head_fold(Wj_ref[...], attnm)], axis=1)
    cA = jnp.concatenate([_bias_fold(bi_ref[...], attnm),
                          _bias_fold(bj_ref[...], attnm)], axis=1)
    aij8 = jnp.dot(e1, wA, preferred_element_type=_f32) + cA
    aij_out[...] = jnp.concatenate([aij8, jnp.zeros((256, 8), _f32)], axis=1)


def _tables_call(e1, skW, skb, Wt, bt, Wi, bi, Wj, bj, attn):
    BQ = 256
    grid = (NPAD // BQ,)
    full = lambda r, c: pl.BlockSpec((r, c), lambda i: (0, 0))
    blk = lambda c: pl.BlockSpec((BQ, c), lambda i: (i, 0))
    return pl.pallas_call(
        _tables_body,
        grid=grid,
        in_specs=[
            blk(D),
            full(O, O), full(1, O),
            full(D, H * O), full(1, H * O),
            full(D, H * O), full(1, H * O),
            full(D, H * O), full(1, H * O),
            full(H, O),
        ],
        out_specs=[blk(D), blk(H * O), blk(16)],
        out_shape=[
            jax.ShapeDtypeStruct((NPAD, D), _f32),
            jax.ShapeDtypeStruct((NPAD, H * O), _f32),
            jax.ShapeDtypeStruct((NPAD, 16), _f32),
        ],
    )(e1, skW, skb.reshape(1, -1), Wt, bt.reshape(1, -1),
      Wi, bi.reshape(1, -1), Wj, bj.reshape(1, -1), attn)


def _final_body(e2_ref, s0_ref, skW_ref, skb_ref, g_ref, b_ref,
                out_ref):
    e2 = e2_ref[...]
    s1 = jnp.dot(e2, skW_ref[...], preferred_element_type=_f32) + skb_ref[...]
    out = e2 + (s0_ref[...] + s1) * 0.5
    mu = jnp.mean(out, axis=1, keepdims=True)
    var = jnp.mean((out - mu) * (out - mu), axis=1, keepdims=True)
    out_ref[...] = (out - mu) * lax.rsqrt(var + 1e-5) * g_ref[...] + b_ref[...]


def _final_call(e2, s0, skW, skb, nf_g, nf_b):
    BQ = 256
    grid = (NPAD // BQ,)
    full = lambda r, c: pl.BlockSpec((r, c), lambda i: (0, 0))
    blk = lambda c: pl.BlockSpec((BQ, c), lambda i: (i, 0))
    return pl.pallas_call(
        _final_body,
        grid=grid,
        in_specs=[blk(D), blk(D), full(O, O), full(1, O),
                  full(1, O), full(1, O)],
        out_specs=blk(D),
        out_shape=jax.ShapeDtypeStruct((NPAD, D), _f32),
    )(e2, s0, skW, skb.reshape(1, -1), nf_g.reshape(1, -1),
      nf_b.reshape(1, -1))


NG = CH // 16


def _mesh():
    return plsc.VectorSubcoreMesh(core_axis_name="c", subcore_axis_name="s",
                                  num_cores=NC, num_subcores=NS)


def _scalar_lane(vec, lane, iota):
    return jnp.max(jnp.where(iota == lane, vec, -3e38))


def _edge_stats_body(src_hbm, dst_hbm, ea_hbm, aijp_hbm,
                     m_hbm, z_hbm, s_hbm,
                     srcv, dstv, eav, pidxd, pidxs, gdp, gsp, row16, sbuf,
                     sem):
    cid = lax.axis_index("c")
    sid = lax.axis_index("s")
    wid = cid * NS + sid
    iota = lax.iota(_i32, 16)

    def chunk_body(c, carry):
        accs = list(carry)
        base = wid * EPW + c * CH
        pltpu.sync_copy(src_hbm.at[pl.ds(base, CH)], srcv)
        pltpu.sync_copy(dst_hbm.at[pl.ds(base, CH)], dstv)
        pltpu.sync_copy(ea_hbm.at[pl.ds(base, CH)], eav)
        for g in range(NG):
            sl = pl.ds(g * 16, 16)
            pidxd[sl] = lax.shift_right_logical(dstv[sl], 3)
            pidxs[sl] = lax.shift_right_logical(srcv[sl], 3)
        cp1 = pltpu.async_copy(aijp_hbm.at[pidxd], gdp, sem)
        cp2 = pltpu.async_copy(aijp_hbm.at[pidxs], gsp, sem)
        cp1.wait()
        cp2.wait()
        for g in range(NG):
            sl = pl.ds(g * 16, 16)
            rows = iota + g * 16
            offd = jnp.bitwise_and(dstv[sl], 7) * 16
            offs = jnp.bitwise_and(srcv[sl], 7) * 16 + 4
            ea16 = eav[sl]
            for h in range(H):
                ga = plsc.load_gather(gdp, [rows, offd + h])
                gj = plsc.load_gather(gsp, [rows, offs + h])
                sh = (ga + gj) * ea16
                sbuf[h, sl] = sh
                m_a, z_a = accs[2 * h], accs[2 * h + 1]
                m_n = jnp.maximum(m_a, sh)
                accs[2 * h] = m_n
                accs[2 * h + 1] = (z_a * jnp.exp(m_a - m_n)
                                   + jnp.exp(sh - m_n))
        pltpu.sync_copy(sbuf, s_hbm.at[wid * NCHUNK + c])
        return tuple(accs)

    init = []
    for _ in range(H):
        init += [jnp.full((16,), -3e38, _f32), jnp.zeros((16,), _f32)]
    accs = lax.fori_loop(0, NCHUNK, chunk_body, tuple(init))

    mvec = jnp.full((16,), -3e38, _f32)
    zvec = jnp.zeros((16,), _f32)
    for h in range(H):
        mh = jnp.max(accs[2 * h])
        zh = jnp.sum(accs[2 * h + 1] * jnp.exp(accs[2 * h] - mh))
        mvec = jnp.where(iota == h, mh, mvec)
        zvec = jnp.where(iota == h, zh, zvec)
    row16[...] = mvec
    pltpu.sync_copy(row16, m_hbm.at[wid])
    row16[...] = zvec
    pltpu.sync_copy(row16, z_hbm.at[wid])


def _edge_stats_call(src, dst, ea, aijp):
    kfn = pl.kernel(
        _edge_stats_body,
        out_type=[jax.ShapeDtypeStruct((NW, 16), _f32),
                  jax.ShapeDtypeStruct((NW, 16), _f32),
                  jax.ShapeDtypeStruct((E // CH, H, CH), _f32)],
        mesh=_mesh(),
        scratch_types=[
            pltpu.VMEM((CH,), _i32),
            pltpu.VMEM((CH,), _i32),
            pltpu.VMEM((CH,), _f32),
            pltpu.VMEM((CH,), _i32),
            pltpu.VMEM((CH,), _i32),
            pltpu.VMEM((CH, 128), _f32),
            pltpu.VMEM((CH, 128), _f32),
            pltpu.VMEM((16,), _f32),
            pltpu.VMEM((H, CH), _f32),
            pltpu.SemaphoreType.DMA,
        ],
        compiler_params=pltpu.CompilerParams(needs_layout_passes=False),
    )
    return kfn(src, dst, ea, aijp)


def _edge_aggr_body(src_hbm, dst_hbm, s_hbm, xtn_hbm,
                    m_hbm, z_hbm, zrows_hbm,
                    part_hbm,
                    srcA, srcB, dstA, dstB, svA, svB, sctA, sctB, pcA, pcB,
                    xrA, xrB, moutv, mzv, abuf, acc_sh,
                    sia, sib, sxa, sxb):
    cid = lax.axis_index("c")
    sid = lax.axis_index("s")
    iota = lax.iota(_i32, 16)
    rowoff = cid * NPAD

    pltpu.sync_copy(zrows_hbm, acc_sh.at[pl.ds(sid * RPS2, RPS2)])

    pltpu.sync_copy(m_hbm, mzv.at[pl.ds(0, NW)])
    pltpu.sync_copy(z_hbm, mzv.at[pl.ds(NW, NW)])

    def mfold(w, m_a):
        return jnp.maximum(m_a, plsc.load_gather(
            mzv, [jnp.full((16,), w, _i32), iota]))
    M = lax.fori_loop(0, NW, mfold, jnp.full((16,), -3e38, _f32))

    def zfold(w, z_a):
        fw = jnp.full((16,), w, _i32)
        mw = plsc.load_gather(mzv, [fw, iota])
        zw = plsc.load_gather(mzv, [fw + NW, iota])
        return z_a + zw * jnp.exp(mw - M)
    Z = lax.fori_loop(0, NW, zfold, jnp.zeros((16,), _f32))

    Mh = [_scalar_lane(M, h, iota) for h in range(H)]
    invZ = jnp.ones((16,), _f32) / jnp.where(iota < H, Z, 1.0)
    iZh = [jnp.max(jnp.where(iota == h, invZ, -3e38)) for h in range(H)]

    plsc.subcore_barrier()

    zv = jnp.zeros((16,), _f32)
    cmax = NCHUNK2 - 1

    def issue_idx(c, srcv, dstv, sv, sem):
        c = jnp.minimum(c, cmax)
        base = sid * EPW2 + c * CH
        j2 = sid * NCHUNK2 + c
        cp1 = pltpu.async_copy(src_hbm.at[pl.ds(base, CH)], srcv, sem)
        cp2 = pltpu.async_copy(dst_hbm.at[pl.ds(base, CH)], dstv, sem)
        cp3 = pltpu.async_copy(s_hbm.at[j2], sv, sem)
        return cp1, cp2, cp3

    def derive(srcv, dstv, sctv, pcv):
        for g in range(NG):
            sl = pl.ds(g * 16, 16)
            d16 = dstv[sl]
            sctv[sl] = lax.shift_right_logical(d16, 1)
            pcv[0, sl] = jnp.bitwise_and(d16, 1) * HCOL
            srcv[sl] = srcv[sl] + rowoff

    def weights(sv):
        for g in range(NG):
            sl = pl.ds(g * 16, 16)
            for h in range(H):
                abuf[h, sl] = jnp.exp(sv[h, sl] - Mh[h]) * iZh[h]

    def run_edges(xrv, pcv, sctv):
        def ebody(e, _e):
            fe = jnp.full((16,), e, _i32)
            cw = plsc.load_gather(pcv, [jnp.zeros((16,), _i32), fe])
            acc = [None] * 4
            for h in range(H):
                w = plsc.load_gather(abuf, [jnp.full((16,), h, _i32), fe])
                for ct in range(4):
                    xv = plsc.load_gather(xrv,
                                          [fe, iota + (h * HCOL + ct * 16)])
                    acc[ct] = w * xv if h == 0 else acc[ct] + w * xv
            for ct in range(4):
                plsc.store_scatter(moutv, [fe, iota + ct * 16 + cw], acc[ct])
                plsc.store_scatter(moutv, [fe, iota + ct * 16 + (HCOL - cw)],
                                   zv)
            return 0

        lax.fori_loop(0, CH, ebody, 0)
        pltpu.sync_copy(moutv, acc_sh.at[sctv], add=True)

    def drain_idx(c, srcv, dstv, sv, sem):
        c = jnp.minimum(c, cmax)
        base = sid * EPW2 + c * CH
        j2 = sid * NCHUNK2 + c
        pltpu.make_async_copy(src_hbm.at[pl.ds(base, CH)], srcv, sem).wait()
        pltpu.make_async_copy(dst_hbm.at[pl.ds(base, CH)], dstv, sem).wait()
        pltpu.make_async_copy(s_hbm.at[j2], sv, sem).wait()

    def drain_xr(srcv, xrv, sem):
        pltpu.make_async_copy(xtn_hbm.at[srcv], xrv, sem).wait()

    for cp in issue_idx(0, srcA, dstA, svA, sia):
        cp.wait()
    derive(srcA, dstA, sctA, pcA)
    pltpu.async_copy(xtn_hbm.at[srcA], xrA, sxa)
    issue_idx(1, srcB, dstB, svB, sib)

    def iter_body(i, _):
        weights(svA)
        drain_idx(2 * i + 1, srcB, dstB, svB, sib)
        derive(srcB, dstB, sctB, pcB)
        pltpu.async_copy(xtn_hbm.at[srcB], xrB, sxb)
        drain_xr(srcA, xrA, sxa)
        run_edges(xrA, pcA, sctA)
        issue_idx(2 * i + 2, srcA, dstA, svA, sia)
        weights(svB)
        drain_idx(2 * i + 2, srcA, dstA, svA, sia)
        derive(srcA, dstA, sctA, pcA)
        pltpu.async_copy(xtn_hbm.at[srcA], xrA, sxa)
        drain_xr(srcB, xrB, sxb)
        run_edges(xrB, pcB, sctB)
        issue_idx(2 * i + 3, srcB, dstB, svB, sib)
        return 0

    lax.fori_loop(0, NCHUNK2 // 2, iter_body, 0)
    drain_idx(NCHUNK2 - 1, srcB, dstB, svB, sib)
    drain_xr(srcA, xrA, sxa)

    plsc.subcore_barrier()
    pltpu.sync_copy(acc_sh.at[pl.ds(sid * RPS2, RPS2)],
                    part_hbm.at[cid, pl.ds(sid * RPS2, RPS2)])


def _edge_aggr_call(src, dst, s_e, xtn, m_p, z_p, zrows):
    kfn = pl.kernel(
        _edge_aggr_body,
        out_type=jax.ShapeDtypeStruct((NC, ACCR, D), _f32),
        mesh=_mesh(),
        scratch_types=[
            pltpu.VMEM((CH,), _i32),
            pltpu.VMEM((CH,), _i32),
            pltpu.VMEM((CH,), _i32),
            pltpu.VMEM((CH,), _i32),
            pltpu.VMEM((H, CH), _f32),
            pltpu.VMEM((H, CH), _f32),
            pltpu.VMEM((CH,), _i32),
            pltpu.VMEM((CH,), _i32),
            pltpu.VMEM((1, CH), _i32),
            pltpu.VMEM((1, CH), _i32),
            pltpu.VMEM((CH, H * HCOL), _f32),
            pltpu.VMEM((CH, H * HCOL), _f32),
            pltpu.VMEM((CH, D), _f32),
            pltpu.VMEM((2 * NW, 16), _f32),
            pltpu.VMEM((H, CH), _f32),
            pltpu.VMEM_SHARED((ACCR, D), _f32),
            pltpu.SemaphoreType.DMA,
            pltpu.SemaphoreType.DMA,
            pltpu.SemaphoreType.DMA,
            pltpu.SemaphoreType.DMA,
        ],
        compiler_params=pltpu.CompilerParams(needs_layout_passes=False),
    )
    return kfn(src, dst, s_e, xtn, m_p, z_p, zrows)


def _edge_layer(src, dst, ea, aij, xtn, zrows):
    aijp = aij.reshape(NPAD // 8, 128)
    xtn2 = jnp.concatenate([xtn[:, :H * HCOL], xtn[:, H * HCOL:]], axis=0)
    m_p, z_p, s_e = _edge_stats_call(src, dst, ea, aijp)
    part = _edge_aggr_call(src, dst, s_e, xtn2, m_p, z_p, zrows)
    return jnp.concatenate([part[0].reshape(NPAD, HCOL),
                            part[1].reshape(NPAD, HCOL)], axis=1)


@jax.jit
def kernel(x, ids, edge_index, edge_attr, embed, qkv_W, qkv_b, proj_W,
           proj_b, lin_W, lin_b, ln_g, ln_b,
           Wi0, bi0, Wj0, bj0, Wt0, bt0, attn0, skW0, skb0,
           Wi1, bi1, Wj1, bj1, Wt1, bt1, attn1, skW1, skb1,
           nf_g, nf_b):
    Ei = jnp.take(embed, ids, axis=0)
    xp = jnp.pad(x, ((0, NPAD - N), (0, 0)))
    embp = jnp.pad(Ei, ((0, NPAD - N), (0, 0)))
    src = edge_index[0].astype(_i32)
    dst = edge_index[1].astype(_i32)
    zrows = jnp.zeros((RPS2, D), _f32)

    perm = jnp.asarray([h * O + hi * HCOL + j
                        for hi in range(2) for h in range(H)
                        for j in range(HCOL)], dtype=_i32)
    Wt0p, bt0p = Wt0[:, perm], bt0[perm]
    Wt1p, bt1p = Wt1[:, perm], bt1[perm]

    q, k, v = _qkv_call(xp, embp, qkv_W, qkv_b)
    emb, xtn0, aij0 = _flash_call(q, k, v, embp, proj_W, proj_b, lin_W, lin_b,
                                  ln_g, ln_b, Wt0p, bt0p, Wi0, bi0, Wj0, bj0,
                                  attn0)
    e1 = _edge_layer(src, dst, edge_attr, aij0, xtn0, zrows)
    s0, xtn1, aij1 = _tables_call(e1, skW0, skb0, Wt1p, bt1p,
                                  Wi1, bi1, Wj1, bj1, attn1)
    e2 = _edge_layer(src, dst, edge_attr, aij1, xtn1, zrows)
    outp = _final_call(e2, s0, skW1, skb1, nf_g, nf_b)
    return outp[:N]
